# baseline (speedup 1.0000x reference)
import jax
import jax.numpy as jnp
from jax import lax
from jax.experimental import pallas as pl
from jax.experimental.pallas import tpu as pltpu
from jax.experimental.pallas import tpu_sc as plsc

B = 16384
D = 32
_INFO = plsc.get_sparse_core_info()
NC, NS, L = _INFO.num_cores, _INFO.num_subcores, _INFO.num_lanes
NW = NC * NS
BPW = B // NW
NG = BPW // L
NSLOT = 8
AHEAD = 7


def _mf_body(*refs):
    (uid_hbm, iid_hbm, pos_hbm, utab_hbm, itab_hbm, ubias_hbm, ibias_hbm,
     gb_hbm, out_hbm) = refs[:9]
    uid_v, iid_v, pos_v, ub_v, ib_v, gb_v, out_v = refs[9:16]
    ustg = refs[16:16 + NSLOT]
    istg = refs[16 + NSLOT:16 + 2 * NSLOT]
    usem = refs[16 + 2 * NSLOT:16 + 3 * NSLOT]
    isem = refs[16 + 3 * NSLOT:16 + 4 * NSLOT]
    sem_ub, sem_ib = refs[16 + 4 * NSLOT:]

    wid = lax.axis_index("s") * NC + lax.axis_index("c")
    base = pl.multiple_of(wid * BPW, BPW)

    pltpu.sync_copy(uid_hbm.at[pl.ds(base, BPW)], uid_v)
    pltpu.sync_copy(iid_hbm.at[pl.ds(base, BPW)], iid_v)
    pltpu.sync_copy(pos_hbm.at[pl.ds(base, BPW)], pos_v)
    pltpu.sync_copy(gb_hbm, gb_v)

    cp_ub = pltpu.make_async_copy(ubias_hbm.at[uid_v], ub_v, sem_ub)
    cp_ib = pltpu.make_async_copy(ibias_hbm.at[iid_v], ib_v, sem_ib)
    cp_ub.start()
    cp_ib.start()

    def copies(ug, ig, slot):
        uo = pl.multiple_of(lax.shift_right_logical(ug, 7) * 128, 128)
        io = pl.multiple_of(lax.shift_right_logical(ig, 7) * 128, 128)
        cu = pltpu.make_async_copy(
            utab_hbm.at[pl.ds(0, D), pl.ds(uo, 128)], ustg[slot], usem[slot])
        ci = pltpu.make_async_copy(
            itab_hbm.at[pl.ds(0, D), pl.ds(io, 128)], istg[slot], isem[slot])
        return cu, ci

    def fire(ug, ig, slot):
        for cp in copies(ug, ig, slot):
            cp.start()

    def drain(ug, ig, slot):
        for cp in copies(ug, ig, slot):
            cp.wait()

    u16p = uid_v[pl.ds(0, L)]
    i16p = iid_v[pl.ds(0, L)]
    for j in range(AHEAD):
        fire(u16p[j], i16p[j], j % NSLOT)

    cp_ub.wait()
    cp_ib.wait()
    gbv = gb_v[...]
    lane = lax.iota(jnp.int32, L)
    lo16 = lax.iota(jnp.int32, L)
    hi16 = lo16 + L
    zero16 = jnp.zeros((L,), jnp.int32)

    def group(blk, carry):
        gbase = pl.multiple_of(blk * L, L)
        u16 = uid_v[pl.ds(gbase, L)]
        i16 = iid_v[pl.ds(gbase, L)]
        dotv = jnp.zeros((L,), jnp.float32)
        for j in range(L):
            ja = j + AHEAD
            if ja < L:
                fire(u16[ja], i16[ja], ja % NSLOT)
            else:
                @pl.when(blk < NG - 1)
                def _():
                    nb = pl.multiple_of((blk + 1) * L, L)
                    u16n = uid_v[pl.ds(nb, L)]
                    i16n = iid_v[pl.ds(nb, L)]
                    fire(u16n[ja - L], i16n[ja - L], ja % NSLOT)
            slot = j % NSLOT
            drain(u16[j], i16[j], slot)
            ucol = zero16 + (u16[j] & 127)
            icol = zero16 + (i16[j] & 127)
            u0 = plsc.load_gather(ustg[slot], [lo16, ucol])
            u1 = plsc.load_gather(ustg[slot], [hi16, ucol])
            v0 = plsc.load_gather(istg[slot], [lo16, icol])
            v1 = plsc.load_gather(istg[slot], [hi16, icol])
            s = jnp.sum(u0 * v0 + u1 * v1)
            dotv = jnp.where(lane == j, s, dotv)
        p = dotv + ub_v[pl.ds(gbase, L)] + ib_v[pl.ds(gbase, L)] + gbv
        p = jnp.minimum(jnp.maximum(p, 1.0), 10.0)
        out_v[pl.ds(gbase, L)] = p
        return carry

    lax.fori_loop(0, NG, group, 0)
    pltpu.make_async_copy(out_v, out_hbm.at[pos_v], sem_ub).start()
    pltpu.make_async_copy(out_v, out_hbm.at[pos_v], sem_ub).wait()


@jax.jit
def _mf(user_ids, item_ids, pos, utab_t, itab_t, ubias_flat, ibias_flat,
        gb16):
    mesh = plsc.VectorSubcoreMesh(core_axis_name="c", subcore_axis_name="s")
    scratch = [
        pltpu.VMEM((BPW,), jnp.int32),
        pltpu.VMEM((BPW,), jnp.int32),
        pltpu.VMEM((BPW,), jnp.int32),
        pltpu.VMEM((BPW,), jnp.float32),
        pltpu.VMEM((BPW,), jnp.float32),
        pltpu.VMEM((L,), jnp.float32),
        pltpu.VMEM((BPW,), jnp.float32),
    ]
    scratch += [pltpu.VMEM((D, 128), jnp.float32) for _ in range(NSLOT)]
    scratch += [pltpu.VMEM((D, 128), jnp.float32) for _ in range(NSLOT)]
    scratch += [pltpu.SemaphoreType.DMA for _ in range(2 * NSLOT)]
    scratch += [pltpu.SemaphoreType.DMA, pltpu.SemaphoreType.DMA]
    run = pl.kernel(
        _mf_body,
        mesh=mesh,
        compiler_params=pltpu.CompilerParams(needs_layout_passes=False),
        out_type=jax.ShapeDtypeStruct((B,), jnp.float32),
        scratch_types=scratch,
    )
    return run(user_ids, item_ids, pos, utab_t, itab_t, ubias_flat,
               ibias_flat, gb16)


def kernel(user_ids, item_ids, user_table, item_table, user_bias, item_bias,
           global_bias):
    utab_t = user_table.T
    itab_t = item_table.T
    ubias_flat = user_bias.reshape(-1)
    ibias_flat = item_bias.reshape(-1)
    gb16 = jnp.broadcast_to(global_bias.astype(jnp.float32), (L,))
    uid = user_ids.astype(jnp.int32)
    iid = item_ids.astype(jnp.int32)
    pos = jnp.argsort(uid).astype(jnp.int32)
    return _mf(uid[pos], iid[pos], pos, utab_t, itab_t, ubias_flat,
               ibias_flat, gb16)


# speedup vs baseline: 1.1445x; 1.1445x over previous
import jax
import jax.numpy as jnp
from jax import lax
from jax.experimental import pallas as pl
from jax.experimental.pallas import tpu as pltpu
from jax.experimental.pallas import tpu_sc as plsc

B = 16384
D = 32
_INFO = plsc.get_sparse_core_info()
NC, NS, L = _INFO.num_cores, _INFO.num_subcores, _INFO.num_lanes
NW = NC * NS
BPW = B // NW
NG = BPW // L
NSLOT = 8
AHEAD = 7


def _mf_body(*refs):
    (uid_hbm, iid_hbm, utab_hbm, itab_hbm, ubias_hbm, ibias_hbm, gb_hbm,
     out_hbm) = refs[:8]
    uid_v, iid_v, ub_v, ib_v, gb_v, out_v = refs[8:14]
    ustg = refs[14:14 + NSLOT]
    istg = refs[14 + NSLOT:14 + 2 * NSLOT]
    usem = refs[14 + 2 * NSLOT:14 + 3 * NSLOT]
    isem = refs[14 + 3 * NSLOT:14 + 4 * NSLOT]
    sem_ub, sem_ib = refs[14 + 4 * NSLOT:]

    wid = lax.axis_index("s") * NC + lax.axis_index("c")
    base = pl.multiple_of(wid * BPW, BPW)

    pltpu.sync_copy(uid_hbm.at[pl.ds(base, BPW)], uid_v)
    pltpu.sync_copy(iid_hbm.at[pl.ds(base, BPW)], iid_v)
    pltpu.sync_copy(gb_hbm, gb_v)

    cp_ub = pltpu.make_async_copy(ubias_hbm.at[uid_v], ub_v, sem_ub)
    cp_ib = pltpu.make_async_copy(ibias_hbm.at[iid_v], ib_v, sem_ib)
    cp_ub.start()
    cp_ib.start()

    def copies(ug, ig, slot):
        uo = pl.multiple_of(lax.shift_right_logical(ug, 7) * 128, 128)
        io = pl.multiple_of(lax.shift_right_logical(ig, 7) * 128, 128)
        cu = pltpu.make_async_copy(
            utab_hbm.at[pl.ds(0, D), pl.ds(uo, 128)], ustg[slot], usem[slot])
        ci = pltpu.make_async_copy(
            itab_hbm.at[pl.ds(0, D), pl.ds(io, 128)], istg[slot], isem[slot])
        return cu, ci

    def fire(ug, ig, slot):
        for cp in copies(ug, ig, slot):
            cp.start()

    def drain(ug, ig, slot):
        for cp in copies(ug, ig, slot):
            cp.wait()

    u16p = uid_v[pl.ds(0, L)]
    i16p = iid_v[pl.ds(0, L)]
    for j in range(AHEAD):
        fire(u16p[j], i16p[j], j % NSLOT)

    cp_ub.wait()
    cp_ib.wait()
    gbv = gb_v[...]
    lane = lax.iota(jnp.int32, L)
    lo16 = lax.iota(jnp.int32, L)
    hi16 = lo16 + L
    zero16 = jnp.zeros((L,), jnp.int32)

    def group(blk, carry):
        gbase = pl.multiple_of(blk * L, L)
        u16 = uid_v[pl.ds(gbase, L)]
        i16 = iid_v[pl.ds(gbase, L)]
        dotv = jnp.zeros((L,), jnp.float32)
        for j in range(L):
            ja = j + AHEAD
            if ja < L:
                fire(u16[ja], i16[ja], ja % NSLOT)
            else:
                @pl.when(blk < NG - 1)
                def _():
                    nb = pl.multiple_of((blk + 1) * L, L)
                    u16n = uid_v[pl.ds(nb, L)]
                    i16n = iid_v[pl.ds(nb, L)]
                    fire(u16n[ja - L], i16n[ja - L], ja % NSLOT)
            slot = j % NSLOT
            drain(u16[j], i16[j], slot)
            ucol = zero16 + (u16[j] & 127)
            icol = zero16 + (i16[j] & 127)
            u0 = plsc.load_gather(ustg[slot], [lo16, ucol])
            u1 = plsc.load_gather(ustg[slot], [hi16, ucol])
            v0 = plsc.load_gather(istg[slot], [lo16, icol])
            v1 = plsc.load_gather(istg[slot], [hi16, icol])
            s = jnp.sum(u0 * v0 + u1 * v1)
            dotv = jnp.where(lane == j, s, dotv)
        p = dotv + ub_v[pl.ds(gbase, L)] + ib_v[pl.ds(gbase, L)] + gbv
        p = jnp.minimum(jnp.maximum(p, 1.0), 10.0)
        out_v[pl.ds(gbase, L)] = p
        return carry

    lax.fori_loop(0, NG, group, 0)
    pltpu.sync_copy(out_v, out_hbm.at[pl.ds(base, BPW)])


@jax.jit
def _mf(user_ids, item_ids, utab_t, itab_t, ubias_flat, ibias_flat, gb16):
    mesh = plsc.VectorSubcoreMesh(core_axis_name="c", subcore_axis_name="s")
    scratch = [
        pltpu.VMEM((BPW,), jnp.int32),
        pltpu.VMEM((BPW,), jnp.int32),
        pltpu.VMEM((BPW,), jnp.float32),
        pltpu.VMEM((BPW,), jnp.float32),
        pltpu.VMEM((L,), jnp.float32),
        pltpu.VMEM((BPW,), jnp.float32),
    ]
    scratch += [pltpu.VMEM((D, 128), jnp.float32) for _ in range(NSLOT)]
    scratch += [pltpu.VMEM((D, 128), jnp.float32) for _ in range(NSLOT)]
    scratch += [pltpu.SemaphoreType.DMA for _ in range(2 * NSLOT)]
    scratch += [pltpu.SemaphoreType.DMA, pltpu.SemaphoreType.DMA]
    run = pl.kernel(
        _mf_body,
        mesh=mesh,
        compiler_params=pltpu.CompilerParams(needs_layout_passes=False),
        out_type=jax.ShapeDtypeStruct((B,), jnp.float32),
        scratch_types=scratch,
    )
    return run(user_ids, item_ids, utab_t, itab_t, ubias_flat, ibias_flat,
               gb16)


def kernel(user_ids, item_ids, user_table, item_table, user_bias, item_bias,
           global_bias):
    utab_t = user_table.T
    itab_t = item_table.T
    ubias_flat = user_bias.reshape(-1)
    ibias_flat = item_bias.reshape(-1)
    gb16 = jnp.broadcast_to(global_bias.astype(jnp.float32), (L,))
    return _mf(user_ids.astype(jnp.int32), item_ids.astype(jnp.int32),
               utab_t, itab_t, ubias_flat, ibias_flat, gb16)
